# baseline (speedup 1.0000x reference)
import functools

import jax
import jax.numpy as jnp
from jax import lax
from jax.experimental import pallas as pl
from jax.experimental.pallas import tpu as pltpu
from jax.experimental.pallas import tpu_sc as plsc

NUM_WORKERS = 32
CHUNK = 32
NBUF = 4


@functools.lru_cache(maxsize=None)
def _embed_call(B, D):
    b_per_w = B // NUM_WORKERS
    n_chunks = b_per_w // CHUNK
    mesh = plsc.VectorSubcoreMesh(core_axis_name="c", subcore_axis_name="s")

    @functools.partial(
        pl.kernel,
        mesh=mesh,
        out_type=jax.ShapeDtypeStruct((B, D), jnp.float32),
        scratch_types=[
            pltpu.VMEM((b_per_w,), jnp.int32),
        ] + [pltpu.VMEM((CHUNK, D), jnp.float32) for _ in range(NBUF)]
          + [pltpu.SemaphoreType.DMA for _ in range(2 * NBUF)],
    )
    def k(tokens_hbm, table_hbm, out_hbm, idx_v, *bufs_and_sems):
        rows = list(bufs_and_sems[:NBUF])
        gsem = list(bufs_and_sems[NBUF:2 * NBUF])
        osem = list(bufs_and_sems[2 * NBUF:3 * NBUF])
        wid = lax.axis_index("s") * 2 + lax.axis_index("c")
        base = wid * b_per_w
        pltpu.sync_copy(tokens_hbm.at[pl.ds(base, b_per_w)], idx_v)
        LOOK = 2
        gather = [None] * n_chunks
        out = [None] * n_chunks
        for c in range(min(LOOK, n_chunks)):
            gather[c] = pltpu.async_copy(
                table_hbm.at[idx_v.at[pl.ds(c * CHUNK, CHUNK)]],
                rows[c % NBUF], gsem[c % NBUF])
        for c in range(n_chunks):
            b = c % NBUF
            gather[c].wait()
            out[c] = pltpu.async_copy(
                rows[b], out_hbm.at[pl.ds(base + c * CHUNK, CHUNK)], osem[b])
            nxt = c + LOOK
            if nxt < n_chunks:
                if nxt >= NBUF:
                    out[nxt - NBUF].wait()
                gather[nxt] = pltpu.async_copy(
                    table_hbm.at[idx_v.at[pl.ds(nxt * CHUNK, CHUNK)]],
                    rows[nxt % NBUF], gsem[nxt % NBUF])
        for c in range(max(0, n_chunks - NBUF), n_chunks):
            out[c].wait()

    return k


def kernel(tokens, W_E):
    batch, seq = tokens.shape
    d_model = W_E.shape[1]
    flat = tokens.reshape(-1).astype(jnp.int32)
    out = _embed_call(batch * seq, d_model)(flat, W_E)
    return out.reshape(batch, seq, d_model)


# speedup vs baseline: 1.0025x; 1.0025x over previous
import functools

import jax
import jax.numpy as jnp
from jax import lax
from jax.experimental import pallas as pl
from jax.experimental.pallas import tpu as pltpu
from jax.experimental.pallas import tpu_sc as plsc

NUM_WORKERS = 32
CHUNK = 64


@functools.lru_cache(maxsize=None)
def _embed_call(batch, seq, D):
    B = batch * seq
    b_per_w = B // NUM_WORKERS
    w_per_row = seq // b_per_w
    n_chunks = b_per_w // CHUNK
    mesh = plsc.VectorSubcoreMesh(core_axis_name="c", subcore_axis_name="s")

    @functools.partial(
        pl.kernel,
        mesh=mesh,
        out_type=jax.ShapeDtypeStruct((batch, seq, D), jnp.float32),
        scratch_types=[
            pltpu.VMEM((b_per_w,), jnp.int32),
            pltpu.VMEM((CHUNK, D), jnp.float32),
            pltpu.VMEM((CHUNK, D), jnp.float32),
            pltpu.SemaphoreType.DMA,
            pltpu.SemaphoreType.DMA,
            pltpu.SemaphoreType.DMA,
            pltpu.SemaphoreType.DMA,
        ],
    )
    def k(tokens_hbm, table_hbm, out_hbm, idx_v, rows0, rows1, gs0, gs1, os0, os1):
        wid = lax.axis_index("s") * 2 + lax.axis_index("c")
        r = wid // w_per_row
        cs = (wid % w_per_row) * b_per_w
        pltpu.sync_copy(tokens_hbm.at[r, pl.ds(cs, b_per_w)], idx_v)
        rows = [rows0, rows1]
        gsem = [gs0, gs1]
        osem = [os0, os1]
        gather = [None] * n_chunks
        out = [None] * n_chunks
        gather[0] = pltpu.async_copy(
            table_hbm.at[idx_v.at[pl.ds(0, CHUNK)]], rows[0], gsem[0])
        for c in range(n_chunks):
            b = c % 2
            if c + 1 < n_chunks:
                nb = (c + 1) % 2
                if c >= 1:
                    out[c - 1].wait()
                gather[c + 1] = pltpu.async_copy(
                    table_hbm.at[idx_v.at[pl.ds((c + 1) * CHUNK, CHUNK)]],
                    rows[nb], gsem[nb])
            gather[c].wait()
            out[c] = pltpu.async_copy(
                rows[b], out_hbm.at[r, pl.ds(cs + c * CHUNK, CHUNK)], osem[b])
        out[n_chunks - 1].wait()
        if n_chunks >= 2:
            out[n_chunks - 2].wait()

    return k


def kernel(tokens, W_E):
    batch, seq = tokens.shape
    d_model = W_E.shape[1]
    return _embed_call(batch, seq, d_model)(tokens, W_E)
